# baseline (speedup 1.0000x reference)
import jax
import jax.numpy as jnp
from jax import lax
from jax.experimental import pallas as pl
from jax.experimental.pallas import tpu as pltpu
from jax.experimental.pallas import tpu_sc as plsc

N_NODES = 10000
N_EDGES = 320000
F = 128

NC = 2
NS = 16
NW = NC * NS
PER_TILE = N_EDGES // NW

CH = 80
NRING = 4


def _zt_body(x_ref, w_ref, b_ref, o_ref):
    o_ref[...] = (
        jnp.dot(x_ref[...], w_ref[...], preferred_element_type=jnp.float32)
        + b_ref[...]
    )


def _edge_body(g_ref, ea_ref, b_ref, o_ref):
    o_ref[...] = g_ref[...] + jnp.dot(
        ea_ref[...], b_ref[...], preferred_element_type=jnp.float32
    )


def _node_body(x_ref, m1_ref, sa_ref, c_ref, d_ref, bd_ref, c3_ref, o_ref):
    m1 = m1_ref[0] + m1_ref[1]
    sa = sa_ref[0] + sa_ref[1]
    o_ref[...] = (
        jnp.dot(x_ref[...], c_ref[...], preferred_element_type=jnp.float32)
        + jnp.dot(m1, d_ref[...], preferred_element_type=jnp.float32)
        + jnp.dot(sa, bd_ref[...], preferred_element_type=jnp.float32)
        + c3_ref[...]
    )


def _gather_m1_body(table_hbm, src_hbm, dst_hbm, zero_hbm, out_hbm, m1_hbm,
                    sidx, didx, rows, acc, sin, sg, swr, ssc):
    c = lax.axis_index("c")
    s = lax.axis_index("s")
    base = (s * NC + c) * PER_TILE
    nring = len(rows)

    @pl.when(s == 0)
    def _():
        pltpu.sync_copy(zero_hbm, acc)

    plsc.subcore_barrier()

    def do_chunk(off, b):
        di = pltpu.async_copy(src_hbm.at[pl.ds(off, CH)], sidx[b], sin[b])
        dd = pltpu.async_copy(dst_hbm.at[pl.ds(off, CH)], didx[b], sin[b])
        di.wait()
        dg = pltpu.async_copy(table_hbm.at[sidx[b]], rows[b], sg[b])
        dg.wait()
        dd.wait()
        dw = pltpu.async_copy(rows[b], out_hbm.at[pl.ds(off, CH)], swr[b])
        ds = pltpu.async_copy(rows[b], acc.at[didx[b]], ssc[b], add=True)
        return dw, ds

    def ring(k, carry):
        waits = []
        for b in range(nring):
            off = base + (nring * k + b) * CH
            waits.append(do_chunk(off, b))
        for dw, ds in waits:
            dw.wait()
            ds.wait()
        return carry

    nring_iters = PER_TILE // (nring * CH)
    lax.fori_loop(0, nring_iters, ring, 0)

    n_tail = (PER_TILE % (nring * CH)) // CH
    for b in range(n_tail):
        off = base + (nring_iters * nring + b) * CH
        dw, ds = do_chunk(off, b)
        dw.wait()
        ds.wait()

    plsc.subcore_barrier()

    @pl.when(s == 0)
    def _():
        pltpu.sync_copy(acc, m1_hbm.at[c])


def _segsum_body(e_hbm, dst_hbm, zero_hbm, out_hbm, idxs, rows, acc, sin, ssc):
    c = lax.axis_index("c")
    s = lax.axis_index("s")
    base = (s * NC + c) * PER_TILE

    @pl.when(s == 0)
    def _():
        pltpu.sync_copy(zero_hbm, acc)

    plsc.subcore_barrier()

    def do_chunk(off, b):
        di = pltpu.async_copy(dst_hbm.at[pl.ds(off, CH)], idxs[b], sin[b])
        dr = pltpu.async_copy(e_hbm.at[pl.ds(off, CH)], rows[b], sin[b])
        di.wait()
        dr.wait()
        return pltpu.async_copy(rows[b], acc.at[idxs[b]], ssc[b], add=True)

    def ring(k, carry):
        waits = []
        for b in range(NRING):
            off = base + (NRING * k + b) * CH
            waits.append(do_chunk(off, b))
        for ds in waits:
            ds.wait()
        return carry

    nring_iters = PER_TILE // (NRING * CH)
    lax.fori_loop(0, nring_iters, ring, 0)

    n_tail = (PER_TILE % (NRING * CH)) // CH
    for b in range(n_tail):
        off = base + (nring_iters * NRING + b) * CH
        do_chunk(off, b).wait()

    plsc.subcore_barrier()

    @pl.when(s == 0)
    def _():
        pltpu.sync_copy(acc, out_hbm.at[c])


def _gather_m1_entry(table, src, dst, zero, out, m1,
                     i0, i1, d0, d1, r0, r1, acc,
                     n0, n1, g0, g1, w0, w1, s0, s1):
    _gather_m1_body(table, src, dst, zero, out, m1,
                    [i0, i1], [d0, d1], [r0, r1],
                    acc, [n0, n1], [g0, g1], [w0, w1], [s0, s1])


def _segsum_entry(e, dst, zero, out,
                  i0, i1, i2, i3, r0, r1, r2, r3, acc,
                  n0, n1, n2, n3, s0, s1, s2, s3):
    _segsum_body(e, dst, zero, out,
                 [i0, i1, i2, i3], [r0, r1, r2, r3], acc,
                 [n0, n1, n2, n3], [s0, s1, s2, s3])


def kernel(x, edge_index, edge_attr, W_nl1, b_nl1, W_el, b_el, W_nm1, b_nm1,
           W_nm2, b_nm2, W_nl2, b_nl2, W_msg, b_msg, W_em, b_em):
    src = edge_index[0]
    dst = edge_index[1]

    M = W_nm1.T @ W_nm2.T
    c2 = b_nm1 @ W_nm2.T + b_nm2
    A1 = W_nl1.T @ M
    Bw = W_el.T @ M
    bias_z = (b_nl1 + b_el) @ M + c2
    Cw = W_nl2.T @ W_em.T
    Dw = W_msg.T @ W_em.T
    BDw = Bw @ Dw
    c3 = (b_nl2 + b_msg) @ W_em.T + b_em

    zt = pl.pallas_call(
        _zt_body,
        out_shape=jax.ShapeDtypeStruct((N_NODES, F), jnp.float32),
    )(x, A1, bias_z[None, :])

    mesh = plsc.VectorSubcoreMesh(core_axis_name="c", subcore_axis_name="s")
    zeros = jnp.zeros((N_NODES, F), jnp.float32)

    gathered, m1parts = pl.kernel(
        _gather_m1_entry,
        out_type=(
            jax.ShapeDtypeStruct((N_EDGES, F), jnp.float32),
            jax.ShapeDtypeStruct((NC, N_NODES, F), jnp.float32),
        ),
        mesh=mesh,
        scratch_types=[pltpu.VMEM((CH,), jnp.int32)] * 4
        + [pltpu.VMEM((CH, F), jnp.float32)] * 2
        + [pltpu.VMEM_SHARED((N_NODES, F), jnp.float32)]
        + [pltpu.SemaphoreType.DMA] * 8,
        name="sc_gather_m1",
    )(zt, src, dst, zeros)

    saparts = pl.kernel(
        _segsum_entry,
        out_type=jax.ShapeDtypeStruct((NC, N_NODES, F), jnp.float32),
        mesh=mesh,
        scratch_types=[pltpu.VMEM((CH,), jnp.int32)] * 4
        + [pltpu.VMEM((CH, F), jnp.float32)] * 4
        + [pltpu.VMEM_SHARED((N_NODES, F), jnp.float32)]
        + [pltpu.SemaphoreType.DMA] * 8,
        name="sc_segsum_ea",
    )(edge_attr, dst, zeros)

    EB = 2000
    e_new = pl.pallas_call(
        _edge_body,
        grid=(N_EDGES // EB,),
        in_specs=[
            pl.BlockSpec((EB, F), lambda i: (i, 0)),
            pl.BlockSpec((EB, F), lambda i: (i, 0)),
            pl.BlockSpec((F, F), lambda i: (0, 0)),
        ],
        out_specs=pl.BlockSpec((EB, F), lambda i: (i, 0)),
        out_shape=jax.ShapeDtypeStruct((N_EDGES, F), jnp.float32),
    )(gathered, edge_attr, Bw)

    NB = 2000
    x_new = pl.pallas_call(
        _node_body,
        grid=(N_NODES // NB,),
        in_specs=[
            pl.BlockSpec((NB, F), lambda i: (i, 0)),
            pl.BlockSpec((NC, NB, F), lambda i: (0, i, 0)),
            pl.BlockSpec((NC, NB, F), lambda i: (0, i, 0)),
            pl.BlockSpec((F, F), lambda i: (0, 0)),
            pl.BlockSpec((F, F), lambda i: (0, 0)),
            pl.BlockSpec((F, F), lambda i: (0, 0)),
            pl.BlockSpec((1, F), lambda i: (0, 0)),
        ],
        out_specs=pl.BlockSpec((NB, F), lambda i: (i, 0)),
        out_shape=jax.ShapeDtypeStruct((N_NODES, F), jnp.float32),
    )(x, m1parts, saparts, Cw, Dw, BDw, c3[None, :])

    return (e_new, x_new)


# speedup vs baseline: 1.0818x; 1.0818x over previous
import jax
import jax.numpy as jnp
from jax import lax
from jax.experimental import pallas as pl
from jax.experimental.pallas import tpu as pltpu
from jax.experimental.pallas import tpu_sc as plsc

N_NODES = 10000
N_EDGES = 320000
F = 128

NC = 2
NS = 16
NW = NC * NS
PER_TILE = N_EDGES // NW

CH = 80


def _zt_body(x_ref, w_ref, b_ref, o_ref):
    o_ref[...] = (
        jnp.dot(x_ref[...], w_ref[...], preferred_element_type=jnp.float32)
        + b_ref[...]
    )


def _q_body(ea_ref, b_ref, o_ref):
    o_ref[...] = jnp.dot(
        ea_ref[...], b_ref[...], preferred_element_type=jnp.float32
    )


def _node_body(x_ref, s_ref, c_ref, d_ref, c3_ref, o_ref):
    msg = s_ref[0] + s_ref[1]
    o_ref[...] = (
        jnp.dot(x_ref[...], c_ref[...], preferred_element_type=jnp.float32)
        + jnp.dot(msg, d_ref[...], preferred_element_type=jnp.float32)
        + c3_ref[...]
    )


def _fused_body(table_hbm, q_hbm, src_hbm, dst_hbm, zero_hbm, e_hbm, msg_hbm,
                sidx, didx, rows, gbuf, acc, sin, sg, swr, ssc):
    c = lax.axis_index("c")
    s = lax.axis_index("s")
    base = (s * NC + c) * PER_TILE
    nring = len(rows)

    @pl.when(s == 0)
    def _():
        pltpu.sync_copy(zero_hbm, acc)

    plsc.subcore_barrier()

    def start_chunk(off, b):
        di = pltpu.async_copy(src_hbm.at[pl.ds(off, CH)], sidx[b], sin[b])
        dd = pltpu.async_copy(dst_hbm.at[pl.ds(off, CH)], didx[b], sin[b])
        dq = pltpu.async_copy(q_hbm.at[pl.ds(off, CH)], rows[b], sin[b])
        di.wait()
        dg = pltpu.async_copy(table_hbm.at[sidx[b]], gbuf[b], sg[b])
        return dd, dq, dg

    def finish_chunk(off, b, dd, dq, dg):
        dq.wait()
        dg.wait()

        def vadd(i, cc):
            for j in range(F // 16):
                sl = pl.ds(j * 16, 16)
                rows[b][i, sl] = rows[b][i, sl] + gbuf[b][i, sl]
            return cc

        lax.fori_loop(0, CH, vadd, 0)
        dd.wait()
        dw = pltpu.async_copy(rows[b], e_hbm.at[pl.ds(off, CH)], swr[b])
        ds = pltpu.async_copy(rows[b], acc.at[didx[b]], ssc[b], add=True)
        return dw, ds

    def ring(k, carry):
        offs = [base + (nring * k + b) * CH for b in range(nring)]
        started = [start_chunk(offs[b], b) for b in range(nring)]
        finished = [
            finish_chunk(offs[b], b, *started[b]) for b in range(nring)
        ]
        for dw, ds in finished:
            dw.wait()
            ds.wait()
        return carry

    nring_iters = PER_TILE // (nring * CH)
    lax.fori_loop(0, nring_iters, ring, 0)

    n_tail = (PER_TILE % (nring * CH)) // CH
    for b in range(n_tail):
        off = base + (nring_iters * nring + b) * CH
        dw, ds = finish_chunk(off, b, *start_chunk(off, b))
        dw.wait()
        ds.wait()

    plsc.subcore_barrier()

    @pl.when(s == 0)
    def _():
        pltpu.sync_copy(acc, msg_hbm.at[c])


def _fused_entry(table, q, src, dst, zero, e, msg,
                 i0, i1, d0, d1, r0, r1, g0, g1, acc,
                 n0, n1, a0, a1, w0, w1, s0, s1):
    _fused_body(table, q, src, dst, zero, e, msg,
                [i0, i1], [d0, d1], [r0, r1], [g0, g1], acc,
                [n0, n1], [a0, a1], [w0, w1], [s0, s1])


def kernel(x, edge_index, edge_attr, W_nl1, b_nl1, W_el, b_el, W_nm1, b_nm1,
           W_nm2, b_nm2, W_nl2, b_nl2, W_msg, b_msg, W_em, b_em):
    src = edge_index[0]
    dst = edge_index[1]

    M = W_nm1.T @ W_nm2.T
    c2 = b_nm1 @ W_nm2.T + b_nm2
    A1 = W_nl1.T @ M
    Bw = W_el.T @ M
    bias_z = (b_nl1 + b_el) @ M + c2
    Cw = W_nl2.T @ W_em.T
    Dw = W_msg.T @ W_em.T
    c3 = (b_nl2 + b_msg) @ W_em.T + b_em

    zt = pl.pallas_call(
        _zt_body,
        out_shape=jax.ShapeDtypeStruct((N_NODES, F), jnp.float32),
    )(x, A1, bias_z[None, :])

    EB = 2000
    q = pl.pallas_call(
        _q_body,
        grid=(N_EDGES // EB,),
        in_specs=[
            pl.BlockSpec((EB, F), lambda i: (i, 0)),
            pl.BlockSpec((F, F), lambda i: (0, 0)),
        ],
        out_specs=pl.BlockSpec((EB, F), lambda i: (i, 0)),
        out_shape=jax.ShapeDtypeStruct((N_EDGES, F), jnp.float32),
    )(edge_attr, Bw)

    mesh = plsc.VectorSubcoreMesh(core_axis_name="c", subcore_axis_name="s")
    zeros = jnp.zeros((N_NODES, F), jnp.float32)

    e_new, msgparts = pl.kernel(
        _fused_entry,
        out_type=(
            jax.ShapeDtypeStruct((N_EDGES, F), jnp.float32),
            jax.ShapeDtypeStruct((NC, N_NODES, F), jnp.float32),
        ),
        mesh=mesh,
        scratch_types=[pltpu.VMEM((CH,), jnp.int32)] * 4
        + [pltpu.VMEM((CH, F), jnp.float32)] * 4
        + [pltpu.VMEM_SHARED((N_NODES, F), jnp.float32)]
        + [pltpu.SemaphoreType.DMA] * 8,
        name="sc_fused_edge",
    )(zt, q, src, dst, zeros)

    NB = 2000
    x_new = pl.pallas_call(
        _node_body,
        grid=(N_NODES // NB,),
        in_specs=[
            pl.BlockSpec((NB, F), lambda i: (i, 0)),
            pl.BlockSpec((NC, NB, F), lambda i: (0, i, 0)),
            pl.BlockSpec((F, F), lambda i: (0, 0)),
            pl.BlockSpec((F, F), lambda i: (0, 0)),
            pl.BlockSpec((1, F), lambda i: (0, 0)),
        ],
        out_specs=pl.BlockSpec((NB, F), lambda i: (i, 0)),
        out_shape=jax.ShapeDtypeStruct((N_NODES, F), jnp.float32),
    )(x, msgparts, Cw, Dw, c3[None, :])

    return (e_new, x_new)


# speedup vs baseline: 1.0820x; 1.0002x over previous
import jax
import jax.numpy as jnp
from jax import lax
from jax.experimental import pallas as pl
from jax.experimental.pallas import tpu as pltpu
from jax.experimental.pallas import tpu_sc as plsc

N_NODES = 10000
N_EDGES = 320000
F = 128

NC = 2
NS = 16
NW = NC * NS
PER_TILE = N_EDGES // NW

CH = 80


def _zt_body(x_ref, w_ref, b_ref, o_ref):
    o_ref[...] = (
        jnp.dot(x_ref[...], w_ref[...], preferred_element_type=jnp.float32)
        + b_ref[...]
    )


def _q_body(ea_ref, b_ref, o_ref):
    o_ref[...] = jnp.dot(
        ea_ref[...], b_ref[...], preferred_element_type=jnp.float32
    )


def _node_body(x_ref, s_ref, c_ref, d_ref, c3_ref, o_ref):
    msg = s_ref[0] + s_ref[1]
    o_ref[...] = (
        jnp.dot(x_ref[...], c_ref[...], preferred_element_type=jnp.float32)
        + jnp.dot(msg, d_ref[...], preferred_element_type=jnp.float32)
        + c3_ref[...]
    )


def _fused_body(table_hbm, q_hbm, src_hbm, dst_hbm, zero_hbm, e_hbm, msg_hbm,
                sidx, didx, rows, gbuf, acc, sin, sg, swr, ssc):
    c = lax.axis_index("c")
    s = lax.axis_index("s")
    base = (s * NC + c) * PER_TILE
    nring = len(rows)

    @pl.when(s == 0)
    def _():
        pltpu.sync_copy(zero_hbm, acc)

    plsc.subcore_barrier()

    def start_chunk(off, b):
        di = pltpu.async_copy(src_hbm.at[pl.ds(off, CH)], sidx[b], sin[b])
        dd = pltpu.async_copy(dst_hbm.at[pl.ds(off, CH)], didx[b], sin[b])
        dq = pltpu.async_copy(q_hbm.at[pl.ds(off, CH)], rows[b], sin[b])
        di.wait()
        dg = pltpu.async_copy(table_hbm.at[sidx[b]], gbuf[b], sg[b])
        return dd, dq, dg

    def finish_chunk(off, b, dd, dq, dg):
        dq.wait()
        dg.wait()

        @plsc.parallel_loop(0, CH, 1, unroll=4)
        def _vadd(i):
            for j in range(F // 16):
                sl = pl.ds(j * 16, 16)
                rows[b][i, sl] = rows[b][i, sl] + gbuf[b][i, sl]
        dd.wait()
        dw = pltpu.async_copy(rows[b], e_hbm.at[pl.ds(off, CH)], swr[b])
        ds = pltpu.async_copy(rows[b], acc.at[didx[b]], ssc[b], add=True)
        return dw, ds

    def ring(k, carry):
        offs = [base + (nring * k + b) * CH for b in range(nring)]
        started = [start_chunk(offs[b], b) for b in range(nring)]
        finished = [
            finish_chunk(offs[b], b, *started[b]) for b in range(nring)
        ]
        for dw, ds in finished:
            dw.wait()
            ds.wait()
        return carry

    nring_iters = PER_TILE // (nring * CH)
    lax.fori_loop(0, nring_iters, ring, 0)

    n_tail = (PER_TILE % (nring * CH)) // CH
    for b in range(n_tail):
        off = base + (nring_iters * nring + b) * CH
        dw, ds = finish_chunk(off, b, *start_chunk(off, b))
        dw.wait()
        ds.wait()

    plsc.subcore_barrier()

    @pl.when(s == 0)
    def _():
        pltpu.sync_copy(acc, msg_hbm.at[c])


def _fused_entry(table, q, src, dst, zero, e, msg,
                 i0, i1, d0, d1, r0, r1, g0, g1, acc,
                 n0, n1, a0, a1, w0, w1, s0, s1):
    _fused_body(table, q, src, dst, zero, e, msg,
                [i0, i1], [d0, d1], [r0, r1], [g0, g1], acc,
                [n0, n1], [a0, a1], [w0, w1], [s0, s1])


def kernel(x, edge_index, edge_attr, W_nl1, b_nl1, W_el, b_el, W_nm1, b_nm1,
           W_nm2, b_nm2, W_nl2, b_nl2, W_msg, b_msg, W_em, b_em):
    src = edge_index[0]
    dst = edge_index[1]

    M = W_nm1.T @ W_nm2.T
    c2 = b_nm1 @ W_nm2.T + b_nm2
    A1 = W_nl1.T @ M
    Bw = W_el.T @ M
    bias_z = (b_nl1 + b_el) @ M + c2
    Cw = W_nl2.T @ W_em.T
    Dw = W_msg.T @ W_em.T
    c3 = (b_nl2 + b_msg) @ W_em.T + b_em

    zt = pl.pallas_call(
        _zt_body,
        out_shape=jax.ShapeDtypeStruct((N_NODES, F), jnp.float32),
    )(x, A1, bias_z[None, :])

    EB = 2000
    q = pl.pallas_call(
        _q_body,
        grid=(N_EDGES // EB,),
        in_specs=[
            pl.BlockSpec((EB, F), lambda i: (i, 0)),
            pl.BlockSpec((F, F), lambda i: (0, 0)),
        ],
        out_specs=pl.BlockSpec((EB, F), lambda i: (i, 0)),
        out_shape=jax.ShapeDtypeStruct((N_EDGES, F), jnp.float32),
    )(edge_attr, Bw)

    mesh = plsc.VectorSubcoreMesh(core_axis_name="c", subcore_axis_name="s")
    zeros = jnp.zeros((N_NODES, F), jnp.float32)

    e_new, msgparts = pl.kernel(
        _fused_entry,
        out_type=(
            jax.ShapeDtypeStruct((N_EDGES, F), jnp.float32),
            jax.ShapeDtypeStruct((NC, N_NODES, F), jnp.float32),
        ),
        mesh=mesh,
        scratch_types=[pltpu.VMEM((CH,), jnp.int32)] * 4
        + [pltpu.VMEM((CH, F), jnp.float32)] * 4
        + [pltpu.VMEM_SHARED((N_NODES, F), jnp.float32)]
        + [pltpu.SemaphoreType.DMA] * 8,
        name="sc_fused_edge",
    )(zt, q, src, dst, zeros)

    NB = 2000
    x_new = pl.pallas_call(
        _node_body,
        grid=(N_NODES // NB,),
        in_specs=[
            pl.BlockSpec((NB, F), lambda i: (i, 0)),
            pl.BlockSpec((NC, NB, F), lambda i: (0, i, 0)),
            pl.BlockSpec((F, F), lambda i: (0, 0)),
            pl.BlockSpec((F, F), lambda i: (0, 0)),
            pl.BlockSpec((1, F), lambda i: (0, 0)),
        ],
        out_specs=pl.BlockSpec((NB, F), lambda i: (i, 0)),
        out_shape=jax.ShapeDtypeStruct((N_NODES, F), jnp.float32),
    )(x, msgparts, Cw, Dw, c3[None, :])

    return (e_new, x_new)
